# baseline (speedup 1.0000x reference)
import jax
import jax.numpy as jnp
from jax.experimental import pallas as pl
from jax.experimental.pallas import tpu as pltpu

_N, _K, _D = 10000, 16, 256
_B = 1000


def _gat_block(self_ref, neigh_ref, lens_ref, w_ref, a2_ref, out_ref):
    x = self_ref[...]
    nb = neigh_ref[...]
    w = w_ref[...]
    a2 = a2_ref[...]
    lens = lens_ref[...]

    wa = jnp.dot(w, a2, preferred_element_type=jnp.float32)
    ex = jnp.dot(x, wa, preferred_element_type=jnp.float32)
    en = jnp.dot(nb, wa, preferred_element_type=jnp.float32)

    e_self = ex[:, 0:1]
    e_neigh = en[:, 1:2].reshape(_B, _K)

    e = e_self + e_neigh
    e = jnp.where(e > 0, e, 0.2 * e)

    valid = jax.lax.broadcasted_iota(jnp.int32, (_B, _K), 1) < jnp.maximum(lens, 1)
    e = jnp.where(valid, e, -1e9)

    m = jnp.max(e, axis=1, keepdims=True)
    p = jnp.exp(e - m)
    alpha = p / jnp.sum(p, axis=1, keepdims=True)

    xagg = x + jnp.sum(alpha[:, :, None] * nb.reshape(_B, _K, _D), axis=1)
    z = jnp.dot(xagg, w, preferred_element_type=jnp.float32)
    out_ref[...] = jnp.where(z > 0, z, jnp.exp(jnp.minimum(z, 0.0)) - 1.0)


def kernel(self_nodes, neigh_nodes, len_adj_nodes, W, a_self, a_neigh):
    neigh2 = neigh_nodes.reshape(_N * _K, _D)
    lens2 = len_adj_nodes.astype(jnp.int32).reshape(_N, 1)
    a2 = jnp.stack([a_self, a_neigh], axis=1)

    grid = (_N // _B,)
    return pl.pallas_call(
        _gat_block,
        grid=grid,
        in_specs=[
            pl.BlockSpec((_B, _D), lambda i: (i, 0)),
            pl.BlockSpec((_B * _K, _D), lambda i: (i, 0)),
            pl.BlockSpec((_B, 1), lambda i: (i, 0)),
            pl.BlockSpec((_D, _D), lambda i: (0, 0)),
            pl.BlockSpec((_D, 2), lambda i: (0, 0)),
        ],
        out_specs=pl.BlockSpec((_B, _D), lambda i: (i, 0)),
        out_shape=jax.ShapeDtypeStruct((_N, _D), jnp.float32),
        compiler_params=pltpu.CompilerParams(
            dimension_semantics=("parallel",),
        ),
    )(self_nodes, neigh2, lens2, W, a2)


# speedup vs baseline: 1.4546x; 1.4546x over previous
import jax
import jax.numpy as jnp
from jax.experimental import pallas as pl
from jax.experimental.pallas import tpu as pltpu

_N, _K, _D = 10000, 16, 256
_B = 1000


def _gat_block(self_ref, neigh_ref, lens_ref, w_ref, a2_ref, out_ref):
    x = self_ref[...]
    nb = neigh_ref[...]
    w = w_ref[...]
    a2 = a2_ref[...]
    lens = lens_ref[...]

    wa = jnp.dot(w, a2, preferred_element_type=jnp.float32)
    wa_s = wa[:, 0]
    wa_n = wa[:, 1]

    e_self = jnp.sum(x * wa_s[None, :], axis=1, keepdims=True)
    e_neigh = jnp.sum(nb.reshape(_B, _K, _D) * wa_n[None, None, :], axis=2)

    e = e_self + e_neigh
    e = jnp.where(e > 0, e, 0.2 * e)

    valid = jax.lax.broadcasted_iota(jnp.int32, (_B, _K), 1) < jnp.maximum(lens, 1)
    e = jnp.where(valid, e, -1e9)

    m = jnp.max(e, axis=1, keepdims=True)
    p = jnp.exp(e - m)
    alpha = p / jnp.sum(p, axis=1, keepdims=True)

    xagg = x + jnp.sum(alpha[:, :, None] * nb.reshape(_B, _K, _D), axis=1)
    z = jnp.dot(xagg, w, preferred_element_type=jnp.float32)
    out_ref[...] = jnp.where(z > 0, z, jnp.exp(jnp.minimum(z, 0.0)) - 1.0)


def kernel(self_nodes, neigh_nodes, len_adj_nodes, W, a_self, a_neigh):
    neigh2 = neigh_nodes.reshape(_N * _K, _D)
    lens2 = len_adj_nodes.astype(jnp.int32).reshape(_N, 1)
    a2 = jnp.stack([a_self, a_neigh], axis=1)

    grid = (_N // _B,)
    return pl.pallas_call(
        _gat_block,
        grid=grid,
        in_specs=[
            pl.BlockSpec((_B, _D), lambda i: (i, 0)),
            pl.BlockSpec((_B * _K, _D), lambda i: (i, 0)),
            pl.BlockSpec((_B, 1), lambda i: (i, 0)),
            pl.BlockSpec((_D, _D), lambda i: (0, 0)),
            pl.BlockSpec((_D, 2), lambda i: (0, 0)),
        ],
        out_specs=pl.BlockSpec((_B, _D), lambda i: (i, 0)),
        out_shape=jax.ShapeDtypeStruct((_N, _D), jnp.float32),
        compiler_params=pltpu.CompilerParams(
            dimension_semantics=("parallel",),
        ),
    )(self_nodes, neigh2, lens2, W, a2)
